# Initial kernel scaffold; baseline (speedup 1.0000x reference)
import jax
import jax.numpy as jnp
from jax.experimental import pallas as pl


def kernel(x, edge_index, W1, b1, W2, b2):
    raise NotImplementedError("write your pallas kernel here")


# speedup vs baseline: 53.0767x; 53.0767x over previous
import functools

import jax
import jax.numpy as jnp
from jax import lax
from jax.experimental import pallas as pl
from jax.experimental.pallas import tpu as pltpu
from jax.experimental.pallas import tpu_sc as plsc

N = 10000
E = 320000
F_IN = 128
H = 16
C = 16

L = 16
NC, NS = 2, 16
NW = NC * NS
EPW = E // NW
CHUNK = 125
NCHUNK = EPW // CHUNK
KFIRE = 8
NBLK = NCHUNK // KFIRE
NPAD = 10240
ORPT = N // NS

_mesh = plsc.VectorSubcoreMesh(core_axis_name="c", subcore_axis_name="s")


SPT = NPAD // NS


@functools.partial(
    pl.kernel,
    out_type=jax.ShapeDtypeStruct((NC, NPAD), jnp.float32),
    mesh=_mesh,
    scratch_types=[
        pltpu.VMEM((EPW,), jnp.int32),
        pltpu.VMEM((NPAD,), jnp.float32),
        pltpu.VMEM((SPT,), jnp.float32),
        pltpu.VMEM((SPT,), jnp.float32),
        pltpu.VMEM_SHARED((NS, NPAD), jnp.float32),
    ],
    compiler_params=pltpu.CompilerParams(
        needs_layout_passes=False, use_tc_tiling_on_sc=False),
)
def _deg_kernel(dst_hbm, out_hbm, dstbuf, pdeg, tmp, accbuf, deg_sh):
    cid = lax.axis_index("c")
    sid = lax.axis_index("s")
    wid = cid * NS + sid
    pltpu.sync_copy(dst_hbm.at[pl.ds(wid * EPW, EPW)], dstbuf)

    zeros16 = jnp.zeros((L,), jnp.float32)

    def zero_body(i, carry):
        pdeg[pl.ds(i * L, L)] = zeros16
        return carry

    lax.fori_loop(0, NPAD // L, zero_body, 0)

    ones16 = jnp.ones((L,), jnp.float32)

    def scat_body(i, carry):
        d = dstbuf[pl.ds(i * L, L)]
        plsc.addupdate_scatter(pdeg, [d], ones16)
        return carry

    lax.fori_loop(0, EPW // L, scat_body, 0)

    pltpu.sync_copy(pdeg, deg_sh.at[sid])
    plsc.subcore_barrier()

    def zacc_body(i, carry):
        accbuf[pl.ds(i * L, L)] = zeros16
        return carry

    lax.fori_loop(0, SPT // L, zacc_body, 0)
    for t in range(NS):
        pltpu.sync_copy(deg_sh.at[t, pl.ds(sid * SPT, SPT)], tmp)

        def add_body(i, carry):
            sl = pl.ds(i * L, L)
            accbuf[sl] = accbuf[sl] + tmp[sl]
            return carry

        lax.fori_loop(0, SPT // L, add_body, 0)
    pltpu.sync_copy(accbuf, out_hbm.at[cid, pl.ds(sid * SPT, SPT)])


@functools.partial(
    pl.kernel,
    out_type=jax.ShapeDtypeStruct((NC, N, L), jnp.float32),
    mesh=_mesh,
    scratch_types=[
        pltpu.VMEM((NCHUNK, CHUNK), jnp.int32),
        pltpu.VMEM((NCHUNK, CHUNK), jnp.int32),
        pltpu.VMEM((KFIRE, CHUNK, L), jnp.float32),
        pltpu.VMEM((ORPT, L), jnp.float32),
        pltpu.VMEM_SHARED((N, L), jnp.float32),
        pltpu.SemaphoreType.DMA,
    ],
    compiler_params=pltpu.CompilerParams(
        needs_layout_passes=False, use_tc_tiling_on_sc=False),
)
def _agg_kernel(y_hbm, src_hbm, dst_hbm, out_hbm, srcbuf, dstbuf, msg, zbuf, acc_sh, sem):
    cid = lax.axis_index("c")
    sid = lax.axis_index("s")
    wid = cid * NS + sid
    pltpu.sync_copy(src_hbm.at[pl.ds(wid * NCHUNK, NCHUNK)], srcbuf)
    pltpu.sync_copy(dst_hbm.at[pl.ds(wid * NCHUNK, NCHUNK)], dstbuf)

    zeros16 = jnp.zeros((L,), jnp.float32)

    def zero_body(i, carry):
        zbuf[i] = zeros16
        return carry

    lax.fori_loop(0, ORPT, zero_body, 0)
    pltpu.sync_copy(zbuf, acc_sh.at[pl.ds(sid * ORPT, ORPT)])
    plsc.subcore_barrier()

    for b in range(KFIRE):
        pltpu.async_copy(y_hbm.at[srcbuf.at[b]], msg.at[b], sem)

    def blk_body(blk, carry):
        for b in range(KFIRE):
            j = blk * KFIRE + b
            pltpu.make_async_copy(y_hbm.at[srcbuf.at[j]], msg.at[b], sem).wait()
            pltpu.sync_copy(msg.at[b], acc_sh.at[dstbuf.at[j]], add=True)

            @pl.when(blk + 1 < NBLK)
            def _():
                pltpu.async_copy(y_hbm.at[srcbuf.at[j + KFIRE]], msg.at[b], sem)

        return carry

    lax.fori_loop(0, NBLK, blk_body, 0)
    plsc.subcore_barrier()
    pltpu.sync_copy(acc_sh.at[pl.ds(sid * ORPT, ORPT)], out_hbm.at[cid, pl.ds(sid * ORPT, ORPT)])


def _tc_a_body(x_ref, w1_ref, p0_ref, p1_ref, y1_ref, dinv_ref):
    deg = p0_ref[...] + p1_ref[...] + 1.0
    dinv = lax.rsqrt(deg)
    xw = jnp.dot(x_ref[...], w1_ref[...], preferred_element_type=jnp.float32)
    y1_ref[...] = xw * dinv
    dinv_ref[...] = dinv


def _tc_b_body(p0_ref, p1_ref, y1_ref, dinv_ref, b1_ref, w2_ref, y2_ref):
    agg = p0_ref[...] + p1_ref[...] + y1_ref[...]
    pre = agg * dinv_ref[...] + b1_ref[...]
    h = jnp.maximum(pre, 0.0)
    hw = jnp.dot(h, w2_ref[...], preferred_element_type=jnp.float32)
    y2_ref[...] = hw * dinv_ref[...]


def _tc_c_body(p0_ref, p1_ref, y2_ref, dinv_ref, b2_ref, out_ref):
    pre = (p0_ref[...] + p1_ref[...] + y2_ref[...]) * dinv_ref[...] + b2_ref[...]
    m = jnp.max(pre, axis=1, keepdims=True)
    ex = jnp.exp(pre - m)
    s = jnp.sum(ex, axis=1, keepdims=True)
    out_ref[...] = pre - m - jnp.log(s)


def kernel(x, edge_index, W1, b1, W2, b2):
    ei = edge_index.astype(jnp.int32)
    src2 = ei[0].reshape(NW * NCHUNK, CHUNK)
    dst2 = ei[1].reshape(NW * NCHUNK, CHUNK)
    dst_flat = ei[1]

    deg_parts = _deg_kernel(dst_flat)
    dp = deg_parts.reshape(NC, NPAD, 1)
    p0d, p1d = dp[0, :N], dp[1, :N]

    y1, dinv = pl.pallas_call(
        _tc_a_body,
        out_shape=[
            jax.ShapeDtypeStruct((N, H), jnp.float32),
            jax.ShapeDtypeStruct((N, 1), jnp.float32),
        ],
    )(x, W1, p0d, p1d)

    parts1 = _agg_kernel(y1, src2, dst2)

    y2 = pl.pallas_call(
        _tc_b_body,
        out_shape=jax.ShapeDtypeStruct((N, C), jnp.float32),
    )(parts1[0], parts1[1], y1, dinv, b1.reshape(1, H), W2)

    parts2 = _agg_kernel(y2, src2, dst2)

    out = pl.pallas_call(
        _tc_c_body,
        out_shape=jax.ShapeDtypeStruct((N, C), jnp.float32),
    )(parts2[0], parts2[1], y2, dinv, b2.reshape(1, C))
    return out
